# baseline (speedup 1.0000x reference)
import functools

import jax
import jax.numpy as jnp
from jax import lax
from jax.experimental import pallas as pl
from jax.experimental.pallas import tpu as pltpu
from jax.experimental.pallas import tpu_sc as plsc


def kernel(indices, values):
    B, H = indices.shape
    V, D = values.shape
    LANES = 128
    HP = 56

    info = plsc.get_sparse_core_info()
    NC, NS = info.num_cores, info.num_subcores
    NW = NC * NS
    b_per_w = B // NW
    n_chunks = b_per_w
    NBUF = 8
    n_outer = n_chunks // NBUF

    idxp = jnp.pad(indices.astype(jnp.int32), ((0, 0), (0, LANES - H)))

    @functools.partial(
        pl.kernel,
        mesh=plsc.VectorSubcoreMesh(core_axis_name="c", subcore_axis_name="s"),
        out_type=jax.ShapeDtypeStruct((B * HP, D), jnp.float32),
        scratch_types=[
            pltpu.VMEM((b_per_w, LANES), jnp.int32),
            pltpu.VMEM((NBUF, HP, D), jnp.float32),
        ]
        + [pltpu.SemaphoreType.DMA] * (2 * NBUF),
        compiler_params=pltpu.CompilerParams(use_tc_tiling_on_sc=False),
    )
    def gather_kernel(table_hbm, idx_hbm, out_hbm, idx_v, rows_v, *sems):
        gsem = sems[:NBUF]
        wsem = sems[NBUF:]
        wid = lax.axis_index("s") * NC + lax.axis_index("c")
        base_b = wid * b_per_w

        def gather_start(i, k):
            pltpu.async_copy(
                table_hbm.at[idx_v.at[i, pl.ds(0, HP)]],
                rows_v.at[k],
                gsem[k],
            )

        def gather_wait(i, k):
            pltpu.make_async_copy(
                table_hbm.at[idx_v.at[i, pl.ds(0, HP)]],
                rows_v.at[k],
                gsem[k],
            ).wait()

        def write_start(i, k):
            pltpu.async_copy(
                rows_v.at[k],
                out_hbm.at[pl.ds((base_b + i) * HP, HP), :],
                wsem[k],
            )

        def write_wait(k):
            pltpu.make_async_copy(
                rows_v.at[k],
                out_hbm.at[pl.ds(base_b * HP, HP), :],
                wsem[k],
            ).wait()

        pltpu.sync_copy(idx_hbm.at[pl.ds(base_b, b_per_w), :], idx_v)

        SLACK = 2
        for k in range(NBUF - SLACK):
            gather_start(k, k)

        def step(i, k, first):
            gather_wait(i, k)
            write_start(i, k)
            gb = (k - SLACK) % NBUF
            if not (first and k < SLACK):
                write_wait(gb)
            gather_start(i + NBUF - SLACK, gb)

        for k in range(NBUF):
            step(k, k, True)

        def outer(o, carry):
            for k in range(NBUF):
                step(o * NBUF + k, k, False)
            return carry

        lax.fori_loop(1, n_outer - 1, outer, 0)

        for k in range(NBUF):
            i = (n_outer - 1) * NBUF + k
            gather_wait(i, k)
            write_start(i, k)
            if k < SLACK:
                gb = (k - SLACK) % NBUF
                write_wait(gb)
                gather_start(i + NBUF - SLACK, gb)
        for k in range(NBUF):
            write_wait(k)

    out = gather_kernel(values, idxp)
    return lax.slice(out.reshape(B, HP, D), (0, 0, 0), (B, H, D))


# speedup vs baseline: 3.1345x; 3.1345x over previous
import functools

import jax
import jax.numpy as jnp
from jax import lax
from jax.experimental import pallas as pl
from jax.experimental.pallas import tpu as pltpu
from jax.experimental.pallas import tpu_sc as plsc


def kernel(indices, values):
    B, H = indices.shape
    V, D = values.shape
    N = B * H
    LANES = 128

    info = plsc.get_sparse_core_info()
    NC, NS = info.num_cores, info.num_subcores
    NW = NC * NS
    n_per_w = N // NW
    C = 128
    n_chunks = n_per_w // C
    NBUF = 10
    n_outer = n_chunks // NBUF

    idxp = jnp.pad(indices.astype(jnp.int32), ((0, 0), (0, LANES - H)))
    idxp = jax.lax.optimization_barrier(idxp)
    idx_flat = idxp.reshape(B * LANES)

    j = jnp.arange(N, dtype=jnp.int32)
    pos3 = ((j // H) * LANES + (j % H)).reshape(NW, n_chunks, C)

    @functools.partial(
        pl.kernel,
        mesh=plsc.VectorSubcoreMesh(core_axis_name="c", subcore_axis_name="s"),
        out_type=jax.ShapeDtypeStruct((N, D), jnp.float32),
        scratch_types=[
            pltpu.VMEM((n_chunks, C), jnp.int32),
            pltpu.VMEM((n_chunks, C), jnp.int32),
            pltpu.VMEM((NBUF, C, D), jnp.float32),
        ]
        + [pltpu.SemaphoreType.DMA] * (1 + 2 * NBUF),
        compiler_params=pltpu.CompilerParams(use_tc_tiling_on_sc=False),
    )
    def gather_kernel(
        table_hbm, idx_hbm, pos_hbm, out_hbm, pos_v, idx_v, rows_v, *sems
    ):
        csem = sems[0]
        gsem = sems[1 : 1 + NBUF]
        wsem = sems[1 + NBUF :]
        wid = lax.axis_index("s") * NC + lax.axis_index("c")
        base = wid * n_per_w

        pltpu.sync_copy(pos_hbm.at[wid], pos_v)

        def compact_start(i, carry):
            pltpu.async_copy(idx_hbm.at[pos_v.at[i]], idx_v.at[i], csem)
            return carry

        def compact_wait(i, carry):
            pltpu.make_async_copy(
                idx_hbm.at[pos_v.at[i]], idx_v.at[i], csem
            ).wait()
            return carry

        lax.fori_loop(0, n_chunks, compact_start, 0)
        lax.fori_loop(0, n_chunks, compact_wait, 0)

        def gather_start(i, k):
            pltpu.async_copy(table_hbm.at[idx_v.at[i]], rows_v.at[k], gsem[k])

        def gather_wait(i, k):
            pltpu.make_async_copy(
                table_hbm.at[idx_v.at[i]], rows_v.at[k], gsem[k]
            ).wait()

        def write_start(i, k):
            pltpu.async_copy(
                rows_v.at[k], out_hbm.at[pl.ds(base + i * C, C)], wsem[k]
            )

        def write_wait(k):
            pltpu.make_async_copy(
                rows_v.at[k], out_hbm.at[pl.ds(base, C)], wsem[k]
            ).wait()

        SLACK = 2
        for k in range(NBUF - SLACK):
            gather_start(k, k)

        def step(i, k, first):
            gather_wait(i, k)
            write_start(i, k)
            gb = (k - SLACK) % NBUF
            if not (first and k < SLACK):
                write_wait(gb)
            gather_start(i + NBUF - SLACK, gb)

        for k in range(NBUF):
            step(k, k, True)

        def outer(o, carry):
            for k in range(NBUF):
                step(o * NBUF + k, k, False)
            return carry

        lax.fori_loop(1, n_outer - 1, outer, 0)

        for k in range(NBUF):
            i = (n_outer - 1) * NBUF + k
            gather_wait(i, k)
            write_start(i, k)
            if k < SLACK:
                gb = (k - SLACK) % NBUF
                write_wait(gb)
                gather_start(i + NBUF - SLACK, gb)
        for k in range(NBUF):
            write_wait(k)

    out = gather_kernel(values, idx_flat, pos3)
    return out.reshape(B, H, D)


# speedup vs baseline: 3.2715x; 1.0437x over previous
import functools

import jax
import jax.numpy as jnp
from jax import lax
from jax.experimental import pallas as pl
from jax.experimental.pallas import tpu as pltpu
from jax.experimental.pallas import tpu_sc as plsc


def kernel(indices, values):
    B, H = indices.shape
    V, D = values.shape
    N = B * H
    LANES = 128

    info = plsc.get_sparse_core_info()
    NC, NS = info.num_cores, info.num_subcores
    NW = NC * NS
    n_per_w = N // NW
    n_batches = B // NW
    n_chunks = n_batches
    NBUF = 8
    n_outer = n_chunks // NBUF

    idx_in = indices.astype(jnp.int32)

    @functools.partial(
        pl.kernel,
        mesh=plsc.VectorSubcoreMesh(core_axis_name="c", subcore_axis_name="s"),
        out_type=jax.ShapeDtypeStruct((N, D), jnp.float32),
        scratch_types=[
            pltpu.VMEM((B // NW, H), jnp.int32),
            pltpu.VMEM((NBUF, H, D), jnp.float32),
        ]
        + [pltpu.SemaphoreType.DMA] * (2 * NBUF),
        compiler_params=pltpu.CompilerParams(use_tc_tiling_on_sc=False),
    )
    def gather_kernel(table_hbm, idx_hbm, out_hbm, idx_v2d, rows_v, *sems):
        gsem = sems[:NBUF]
        wsem = sems[NBUF:]
        wid = lax.axis_index("s") * NC + lax.axis_index("c")
        base = wid * n_per_w

        pltpu.sync_copy(
            idx_hbm.at[pl.ds(wid * n_batches, n_batches), :], idx_v2d
        )

        def gather_start(i, k):
            pltpu.async_copy(
                table_hbm.at[idx_v2d.at[i]], rows_v.at[k], gsem[k]
            )

        def gather_wait(i, k):
            pltpu.make_async_copy(
                table_hbm.at[idx_v2d.at[i]], rows_v.at[k], gsem[k]
            ).wait()

        def write_start(i, k):
            pltpu.async_copy(
                rows_v.at[k], out_hbm.at[pl.ds(base + i * H, H)], wsem[k]
            )

        def write_wait(k):
            pltpu.make_async_copy(
                rows_v.at[k], out_hbm.at[pl.ds(base, H)], wsem[k]
            ).wait()

        SLACK = 2
        for k in range(NBUF - SLACK):
            gather_start(k, k)

        def step(i, k, first):
            gather_wait(i, k)
            write_start(i, k)
            gb = (k - SLACK) % NBUF
            if not (first and k < SLACK):
                write_wait(gb)
            gather_start(i + NBUF - SLACK, gb)

        for k in range(NBUF):
            step(k, k, True)

        def outer(o, carry):
            for k in range(NBUF):
                step(o * NBUF + k, k, False)
            return carry

        lax.fori_loop(1, n_outer - 1, outer, 0)

        for k in range(NBUF):
            i = (n_outer - 1) * NBUF + k
            gather_wait(i, k)
            write_start(i, k)
            if k < SLACK:
                gb = (k - SLACK) % NBUF
                write_wait(gb)
                gather_start(i + NBUF - SLACK, gb)
        for k in range(NBUF):
            write_wait(k)

    out = gather_kernel(values, idx_in)
    return out.reshape(B, H, D)
